# baseline (speedup 1.0000x reference)
import functools

import jax
import jax.numpy as jnp
from jax import lax
from jax.experimental import pallas as pl
from jax.experimental.pallas import tpu as pltpu
from jax.experimental.pallas import tpu_sc as plsc

_N = 4096 * 32 * 32
_NUM_WORKERS = 32
_PER_WORKER = _N // _NUM_WORKERS
_CHUNK = 8192
_CHUNKS = _PER_WORKER // _CHUNK
_LANES = 16


def _gate_body(ent_hbm, tf_hbm, tm_hbm, out_hbm, in_v, out_v, tf_v, tm_v):
    ncores = 2
    wid = lax.axis_index("s") * ncores + lax.axis_index("c")
    base = wid * _PER_WORKER

    pltpu.sync_copy(tf_hbm, tf_v)
    pltpu.sync_copy(tm_hbm, tm_v)
    tf = tf_v[...]
    tm = tm_v[...]
    lane3 = lax.iota(jnp.int32, _LANES) * 3

    for g in range(_CHUNKS):
        in_off = base + g * _CHUNK
        pltpu.sync_copy(ent_hbm.at[pl.ds(in_off, _CHUNK)], in_v)

        @plsc.parallel_loop(0, _CHUNK // _LANES, unroll=8)
        def _(k):
            off = pl.multiple_of(k * _LANES, _LANES)
            v = in_v[pl.ds(off, _LANES)]
            coarse = (v <= tm).astype(jnp.int32)
            med = ((v <= tf) & (v > tm)).astype(jnp.int32)
            fine = (v > tf).astype(jnp.int32)
            idx = k * (3 * _LANES) + lane3
            plsc.store_scatter(out_v, [idx], coarse)
            plsc.store_scatter(out_v, [idx + 1], med)
            plsc.store_scatter(out_v, [idx + 2], fine)

        pltpu.sync_copy(out_v, out_hbm.at[pl.ds(3 * in_off, 3 * _CHUNK)])


def kernel(entropy, threshold_fine, threshold_median):
    ent = entropy.reshape(_N)
    tf = jnp.full((_LANES,), threshold_fine, jnp.float32)
    tm = jnp.full((_LANES,), threshold_median, jnp.float32)
    mesh = plsc.VectorSubcoreMesh(core_axis_name="c", subcore_axis_name="s")
    run = functools.partial(
        pl.kernel,
        out_type=jax.ShapeDtypeStruct((3 * _N,), jnp.int32),
        mesh=mesh,
        compiler_params=pltpu.CompilerParams(needs_layout_passes=False),
        scratch_types=[
            pltpu.VMEM((_CHUNK,), jnp.float32),
            pltpu.VMEM((3 * _CHUNK,), jnp.int32),
            pltpu.VMEM((_LANES,), jnp.float32),
            pltpu.VMEM((_LANES,), jnp.float32),
        ],
    )(_gate_body)
    out = run(ent, tf, tm)
    return out.reshape(4096, 32, 32, 3)


# speedup vs baseline: 39.2117x; 39.2117x over previous
import functools

import jax
import jax.numpy as jnp
from jax import lax
from jax.experimental import pallas as pl
from jax.experimental.pallas import tpu as pltpu
from jax.experimental.pallas import tpu_sc as plsc

_B = 4096
_H = 32
_W = 32
_LANES = 16
_IN_ROWS = _H * _W * _B // 128
_ROWS_PW = _IN_ROWS // 32
_CROWS = 64
_CHUNKS = _ROWS_PW // _CROWS


def _gate_body(ent_hbm, tf_hbm, tm_hbm, out_hbm, in_v, c_v, m_v, f_v,
               tf_v, tm_v):
    ncores = 2
    wid = lax.axis_index("s") * ncores + lax.axis_index("c")

    pltpu.sync_copy(tf_hbm, tf_v)
    pltpu.sync_copy(tm_hbm, tm_v)
    tf = tf_v[...]
    tm = tm_v[...]

    in_base = wid * _ROWS_PW
    out_base = wid * (3 * _ROWS_PW)

    for g in range(_CHUNKS):
        r0 = in_base + g * _CROWS
        pltpu.sync_copy(ent_hbm.at[pl.ds(r0, _CROWS)], in_v)

        @plsc.parallel_loop(0, _CROWS * (128 // _LANES), unroll=8)
        def _(k):
            r = k // (128 // _LANES)
            c = pl.multiple_of((k % (128 // _LANES)) * _LANES, _LANES)
            v = in_v[r, pl.ds(c, _LANES)]
            c_v[r, pl.ds(c, _LANES)] = (v <= tm).astype(jnp.int32)
            m_v[r, pl.ds(c, _LANES)] = ((v <= tf) & (v > tm)).astype(jnp.int32)
            f_v[r, pl.ds(c, _LANES)] = (v > tf).astype(jnp.int32)

        local = g * _CROWS
        pltpu.sync_copy(c_v, out_hbm.at[pl.ds(out_base + local, _CROWS)])
        pltpu.sync_copy(m_v, out_hbm.at[pl.ds(out_base + _ROWS_PW + local, _CROWS)])
        pltpu.sync_copy(f_v, out_hbm.at[pl.ds(out_base + 2 * _ROWS_PW + local, _CROWS)])


def kernel(entropy, threshold_fine, threshold_median):
    e = jnp.transpose(entropy, (1, 2, 0))
    e = e.reshape(_H, _W // 8, 8, _B // 128, 128)
    e = jnp.transpose(e, (0, 1, 3, 2, 4))
    ent_lin = e.reshape(_IN_ROWS, 128)

    tf = jnp.full((_LANES,), threshold_fine, jnp.float32)
    tm = jnp.full((_LANES,), threshold_median, jnp.float32)

    mesh = plsc.VectorSubcoreMesh(core_axis_name="c", subcore_axis_name="s")
    run = functools.partial(
        pl.kernel,
        out_type=jax.ShapeDtypeStruct((3 * _IN_ROWS, 128), jnp.int32),
        mesh=mesh,
        compiler_params=pltpu.CompilerParams(needs_layout_passes=False),
        scratch_types=[
            pltpu.VMEM((_CROWS, 128), jnp.float32),
            pltpu.VMEM((_CROWS, 128), jnp.int32),
            pltpu.VMEM((_CROWS, 128), jnp.int32),
            pltpu.VMEM((_CROWS, 128), jnp.int32),
            pltpu.VMEM((_LANES,), jnp.float32),
            pltpu.VMEM((_LANES,), jnp.float32),
        ],
    )(_gate_body)
    out = run(ent_lin, tf, tm)

    o = out.reshape(_H, 3, _W // 8, _B // 128, 8, 128)
    o = jnp.transpose(o, (3, 5, 0, 2, 4, 1))
    return o.reshape(_B, _H, _W, 3)


# speedup vs baseline: 56.8139x; 1.4489x over previous
import functools

import jax
import jax.numpy as jnp
from jax import lax
from jax.experimental import pallas as pl
from jax.experimental.pallas import tpu as pltpu
from jax.experimental.pallas import tpu_sc as plsc

_B = 4096
_H = 32
_W = 32
_LANES = 16
_IN_ROWS = _H * _W * _B // 128
_ROWS_PW = _IN_ROWS // 32
_CROWS = 64
_CHUNKS = _ROWS_PW // _CROWS


def _gate_body(ent_hbm, tf_hbm, tm_hbm, out_hbm,
               in_v0, in_v1, c_v0, m_v0, f_v0, c_v1, m_v1, f_v1,
               tf_v, tm_v, sin0, sin1, sout0, sout1):
    ncores = 2
    wid = lax.axis_index("s") * ncores + lax.axis_index("c")

    pltpu.sync_copy(tf_hbm, tf_v)
    pltpu.sync_copy(tm_hbm, tm_v)
    tf = tf_v[...]
    tm = tm_v[...]

    in_base = wid * _ROWS_PW
    out_base = wid * (3 * _ROWS_PW)

    in_bufs = (in_v0, in_v1)
    out_bufs = ((c_v0, m_v0, f_v0), (c_v1, m_v1, f_v1))
    in_sems = (sin0, sin1)
    out_sems = (sout0, sout1)

    def start_in(g):
        p = g & 1
        return pltpu.async_copy(
            ent_hbm.at[pl.ds(in_base + g * _CROWS, _CROWS)], in_bufs[p],
            in_sems[p])

    def start_out(g):
        p = g & 1
        local = g * _CROWS
        return [
            pltpu.async_copy(
                out_bufs[p][j],
                out_hbm.at[pl.ds(out_base + j * _ROWS_PW + local, _CROWS)],
                out_sems[p])
            for j in range(3)
        ]

    h_in = {0: start_in(0)}
    h_out = {}
    for g in range(_CHUNKS):
        p = g & 1
        h_in.pop(g).wait()
        if g + 1 < _CHUNKS:
            h_in[g + 1] = start_in(g + 1)
        if g >= 2:
            for h in h_out.pop(g - 2):
                h.wait()

        cb, mb, fb = out_bufs[p]
        ib = in_bufs[p]

        @plsc.parallel_loop(0, _CROWS * (128 // _LANES), unroll=8)
        def _(k):
            r = k // (128 // _LANES)
            c = pl.multiple_of((k % (128 // _LANES)) * _LANES, _LANES)
            v = ib[r, pl.ds(c, _LANES)]
            cb[r, pl.ds(c, _LANES)] = (v <= tm).astype(jnp.int32)
            mb[r, pl.ds(c, _LANES)] = ((v <= tf) & (v > tm)).astype(jnp.int32)
            fb[r, pl.ds(c, _LANES)] = (v > tf).astype(jnp.int32)

        h_out[g] = start_out(g)

    for g in (_CHUNKS - 2, _CHUNKS - 1):
        for h in h_out.pop(g):
            h.wait()


def kernel(entropy, threshold_fine, threshold_median):
    e = jnp.transpose(entropy, (1, 2, 0))
    e = e.reshape(_H, _W // 8, 8, _B // 128, 128)
    e = jnp.transpose(e, (0, 1, 3, 2, 4))
    ent_lin = e.reshape(_IN_ROWS, 128)

    tf = jnp.full((_LANES,), threshold_fine, jnp.float32)
    tm = jnp.full((_LANES,), threshold_median, jnp.float32)

    mesh = plsc.VectorSubcoreMesh(core_axis_name="c", subcore_axis_name="s")
    run = functools.partial(
        pl.kernel,
        out_type=jax.ShapeDtypeStruct((3 * _IN_ROWS, 128), jnp.int32),
        mesh=mesh,
        compiler_params=pltpu.CompilerParams(needs_layout_passes=False),
        scratch_types=[
            pltpu.VMEM((_CROWS, 128), jnp.float32),
            pltpu.VMEM((_CROWS, 128), jnp.float32),
            pltpu.VMEM((_CROWS, 128), jnp.int32),
            pltpu.VMEM((_CROWS, 128), jnp.int32),
            pltpu.VMEM((_CROWS, 128), jnp.int32),
            pltpu.VMEM((_CROWS, 128), jnp.int32),
            pltpu.VMEM((_CROWS, 128), jnp.int32),
            pltpu.VMEM((_CROWS, 128), jnp.int32),
            pltpu.VMEM((_LANES,), jnp.float32),
            pltpu.VMEM((_LANES,), jnp.float32),
            pltpu.SemaphoreType.DMA,
            pltpu.SemaphoreType.DMA,
            pltpu.SemaphoreType.DMA,
            pltpu.SemaphoreType.DMA,
        ],
    )(_gate_body)
    out = run(ent_lin, tf, tm)

    o = out.reshape(_H, 3, _W // 8, _B // 128, 8, 128)
    o = jnp.transpose(o, (3, 5, 0, 2, 4, 1))
    return o.reshape(_B, _H, _W, 3)
